# Initial kernel scaffold; baseline (speedup 1.0000x reference)
import jax
import jax.numpy as jnp
from jax.experimental import pallas as pl


def kernel(x, adjs, W_gc, b_gc, W_mlp, b_mlp, W_cls, b_cls):
    raise NotImplementedError("write your pallas kernel here")


# speedup vs baseline: 1.6066x; 1.6066x over previous
import functools

import jax
import jax.numpy as jnp
from jax.experimental import pallas as pl

N = 4096
FEAT = 128
HID = 16
NH = 9
NC = 27
BLK = 512


def _support_body(x_ref, w_ref, out_ref):
    out_ref[0] = jnp.dot(x_ref[0], w_ref[0], preferred_element_type=jnp.float32)


def _main_body(adj_ref, sup_ref, b_gc_ref, w_mlp_ref, b_mlp_ref, w_cls_ref,
               b_cls_ref, out_ref):
    i = pl.program_id(1)
    adj = adj_ref[0]
    adj = jnp.where(jnp.isnan(adj), 0.0, adj)
    h = jnp.dot(adj, sup_ref[0], preferred_element_type=jnp.float32)
    h = h + b_gc_ref[i]
    t = jnp.tanh(jnp.dot(h, w_mlp_ref[i], preferred_element_type=jnp.float32)
                 + b_mlp_ref[i])
    w_cls_i = w_cls_ref[pl.ds(i * NH, NH), :]
    contrib = jnp.dot(t, w_cls_i, preferred_element_type=jnp.float32)

    @pl.when(i == 0)
    def _():
        out_ref[...] = contrib + b_cls_ref[0]

    @pl.when(i != 0)
    def _():
        out_ref[...] += contrib


@jax.jit
def kernel(x, adjs, W_gc, b_gc, W_mlp, b_mlp, W_cls, b_cls):
    support = pl.pallas_call(
        _support_body,
        grid=(3,),
        in_specs=[
            pl.BlockSpec((1, N, FEAT), lambda i: (i, 0, 0)),
            pl.BlockSpec((1, FEAT, HID), lambda i: (i, 0, 0)),
        ],
        out_specs=pl.BlockSpec((1, N, HID), lambda i: (i, 0, 0)),
        out_shape=jax.ShapeDtypeStruct((3, N, HID), jnp.float32),
    )(x, W_gc)

    nb = N // BLK
    out = pl.pallas_call(
        _main_body,
        grid=(nb, 3),
        in_specs=[
            pl.BlockSpec((1, BLK, N), lambda b, i: (i, b, 0)),
            pl.BlockSpec((1, N, HID), lambda b, i: (i, 0, 0)),
            pl.BlockSpec((3, HID), lambda b, i: (0, 0)),
            pl.BlockSpec((3, HID, NH), lambda b, i: (0, 0, 0)),
            pl.BlockSpec((3, NH), lambda b, i: (0, 0)),
            pl.BlockSpec((3 * NH, NC), lambda b, i: (0, 0)),
            pl.BlockSpec((1, NC), lambda b, i: (0, 0)),
        ],
        out_specs=pl.BlockSpec((BLK, NC), lambda b, i: (b, 0)),
        out_shape=jax.ShapeDtypeStruct((N, NC), jnp.float32),
    )(adjs, support, b_gc, W_mlp, b_mlp, W_cls, b_cls.reshape(1, NC))
    return out
